# Initial kernel scaffold; baseline (speedup 1.0000x reference)
import jax
import jax.numpy as jnp
from jax.experimental import pallas as pl


def kernel(x, edge_index, W_edge, b_edge, W_gcn, b_gcn):
    raise NotImplementedError("write your pallas kernel here")


# speedup vs baseline: 27.3312x; 27.3312x over previous
import functools

import jax
import jax.numpy as jnp
from jax import lax
from jax.experimental import pallas as pl
from jax.experimental.pallas import tpu as pltpu
from jax.experimental.pallas import tpu_sc as plsc

NC = 2
NS = 16
NW = NC * NS
L = 16


def _make_hist(E, N):
    epw = E // NW
    mesh = plsc.VectorSubcoreMesh(core_axis_name="c", subcore_axis_name="s")

    @functools.partial(
        pl.kernel,
        mesh=mesh,
        out_type=jax.ShapeDtypeStruct((NW, N), jnp.float32),
        compiler_params=pltpu.CompilerParams(needs_layout_passes=False),
        scratch_types=[
            pltpu.VMEM((epw,), jnp.int32),
            pltpu.VMEM((N,), jnp.float32),
        ],
    )
    def hist(row_hbm, degp_hbm, row_v, acc_v):
        wid = lax.axis_index("s") * NC + lax.axis_index("c")
        base = wid * epw
        pltpu.sync_copy(row_hbm.at[pl.ds(base, epw)], row_v)

        zeros = jnp.zeros((L,), jnp.float32)

        def zbody(i, c):
            acc_v[pl.ds(i * L, L)] = zeros
            return c

        lax.fori_loop(0, N // L, zbody, 0)

        ones = jnp.ones((L,), jnp.float32)

        def body(i, c):
            idx = row_v[pl.ds(i * L, L)]
            plsc.addupdate_scatter(acc_v, [idx], ones)
            return c

        lax.fori_loop(0, epw // L, body, 0)
        pltpu.sync_copy(acc_v, degp_hbm.at[wid])

    return hist


def _make_edges(E, N):
    epw = E // NW
    mesh = plsc.VectorSubcoreMesh(core_axis_name="c", subcore_axis_name="s")

    @functools.partial(
        pl.kernel,
        mesh=mesh,
        out_type=(
            jax.ShapeDtypeStruct((2 * E,), jnp.float32),
            jax.ShapeDtypeStruct((2, NW, N), jnp.float32),
        ),
        compiler_params=pltpu.CompilerParams(needs_layout_passes=False),
        scratch_types=[
            pltpu.VMEM((epw,), jnp.int32),
            pltpu.VMEM((epw,), jnp.int32),
            pltpu.VMEM((N,), jnp.float32),
            pltpu.VMEM((N,), jnp.float32),
            pltpu.VMEM((N,), jnp.float32),
            pltpu.VMEM((N,), jnp.float32),
            pltpu.VMEM((2 * epw,), jnp.float32),
            pltpu.VMEM((N,), jnp.float32),
            pltpu.VMEM((N,), jnp.float32),
        ],
    )
    def edges(row_hbm, col_hbm, tab_hbm, att_hbm, sp_hbm,
              row_v, col_v, u_v, v_v, g0_v, g1_v, att_v, s0_v, s1_v):
        wid = lax.axis_index("s") * NC + lax.axis_index("c")
        base = wid * epw
        pltpu.sync_copy(row_hbm.at[pl.ds(base, epw)], row_v)
        pltpu.sync_copy(col_hbm.at[pl.ds(base, epw)], col_v)
        pltpu.sync_copy(tab_hbm.at[0], u_v)
        pltpu.sync_copy(tab_hbm.at[1], v_v)
        pltpu.sync_copy(tab_hbm.at[2], g0_v)
        pltpu.sync_copy(tab_hbm.at[3], g1_v)

        zeros = jnp.zeros((L,), jnp.float32)

        def zbody(i, c):
            s0_v[pl.ds(i * L, L)] = zeros
            s1_v[pl.ds(i * L, L)] = zeros
            return c

        lax.fori_loop(0, N // L, zbody, 0)

        ii2 = lax.iota(jnp.int32, L) * 2

        def body(i, c):
            idxr = row_v[pl.ds(i * L, L)]
            idxc = col_v[pl.ds(i * L, L)]
            ug = plsc.load_gather(u_v, [idxr])
            vg = plsc.load_gather(v_v, [idxc])
            d = ug + vg
            e = jnp.exp(d)
            a0 = 1.0 / (1.0 + e)
            a1 = e * a0
            pos = ii2 + (i * (2 * L))
            plsc.store_scatter(att_v, [pos], a0)
            plsc.store_scatter(att_v, [pos + 1], a1)
            gr0 = plsc.load_gather(g0_v, [idxr])
            gr1 = plsc.load_gather(g1_v, [idxr])
            plsc.addupdate_scatter(s0_v, [idxc], gr0)
            plsc.addupdate_scatter(s1_v, [idxc], gr1)
            return c

        lax.fori_loop(0, epw // L, body, 0)

        pltpu.sync_copy(att_v, att_hbm.at[pl.ds(2 * base, 2 * epw)])
        pltpu.sync_copy(s0_v, sp_hbm.at[0, wid])
        pltpu.sync_copy(s1_v, sp_hbm.at[1, wid])

    return edges


def _tables_body(x_ref, wcat_ref, consts_ref, degp_ref, tab_ref, fin_ref):
    x = x_ref[...]
    wcat = wcat_ref[...]
    ht = lax.dot_general(wcat, x, (((0,), (1,)), ((), ())),
                         preferred_element_type=jnp.float32)
    c0 = consts_ref[0]
    bg0 = consts_ref[1]
    bg1 = consts_ref[2]
    deg = jnp.sum(degp_ref[...], axis=0, keepdims=True) + 1.0
    dis = lax.rsqrt(deg)
    u = ht[1:2] - ht[0:1] + c0
    v = ht[3:4] - ht[2:3]
    gp0 = dis * ht[4:5]
    gp1 = dis * ht[5:6]
    base0 = dis * gp0 + bg0
    base1 = dis * gp1 + bg1
    tab_ref[...] = jnp.concatenate([u, v, gp0, gp1], axis=0)
    fin_ref[...] = jnp.concatenate([dis, base0, base1], axis=0)


def _final_body(fin_ref, sp_ref, out_ref):
    dis = fin_ref[0:1]
    base0 = fin_ref[1:2]
    base1 = fin_ref[2:3]
    s0 = jnp.sum(sp_ref[0:NW], axis=0, keepdims=True)
    s1 = jnp.sum(sp_ref[NW:2 * NW], axis=0, keepdims=True)
    l0 = dis * s0 + base0
    l1 = dis * s1 + base1
    e = jnp.exp(l1 - l0)
    n0 = 1.0 / (1.0 + e)
    n1 = e * n0
    out_ref[...] = jnp.concatenate([n0, n1], axis=0)


def kernel(x, edge_index, W_edge, b_edge, W_gcn, b_gcn):
    N, D = x.shape
    E = edge_index.shape[1]

    wcat = jnp.concatenate([W_edge[:D], W_edge[D:], W_gcn], axis=1)
    consts = jnp.stack([b_edge[1] - b_edge[0], b_gcn[0], b_gcn[1]])

    row = edge_index[0]
    col = edge_index[1]
    degp = _make_hist(E, N)(row)

    tab, fin = pl.pallas_call(
        _tables_body,
        out_shape=(
            jax.ShapeDtypeStruct((4, N), jnp.float32),
            jax.ShapeDtypeStruct((3, N), jnp.float32),
        ),
        in_specs=[
            pl.BlockSpec(memory_space=pltpu.VMEM),
            pl.BlockSpec(memory_space=pltpu.VMEM),
            pl.BlockSpec(memory_space=pltpu.SMEM),
            pl.BlockSpec(memory_space=pltpu.VMEM),
        ],
    )(x, wcat, consts, degp)

    att_flat, sp = _make_edges(E, N)(row, col, tab)

    out2 = pl.pallas_call(
        _final_body,
        out_shape=jax.ShapeDtypeStruct((2, N), jnp.float32),
    )(fin, sp.reshape(2 * NW, N))

    return att_flat.reshape(E, 2), out2.T


# speedup vs baseline: 107.5437x; 3.9348x over previous
import functools

import jax
import jax.numpy as jnp
from jax import lax
from jax.experimental import pallas as pl
from jax.experimental.pallas import tpu as pltpu
from jax.experimental.pallas import tpu_sc as plsc

NC = 2
NS = 16
NW = NC * NS
L = 16


def _make_hist(E, N):
    epw = E // NW
    mesh = plsc.VectorSubcoreMesh(core_axis_name="c", subcore_axis_name="s")

    @functools.partial(
        pl.kernel,
        mesh=mesh,
        out_type=jax.ShapeDtypeStruct((NW, N), jnp.float32),
        compiler_params=pltpu.CompilerParams(needs_layout_passes=False),
        scratch_types=[
            pltpu.VMEM((epw,), jnp.int32),
            pltpu.VMEM((N,), jnp.float32),
        ],
    )
    def hist(row_hbm, degp_hbm, row_v, acc_v):
        wid = lax.axis_index("s") * NC + lax.axis_index("c")
        base = wid * epw
        pltpu.sync_copy(row_hbm.at[pl.ds(base, epw)], row_v)

        zeros = jnp.zeros((L,), jnp.float32)

        def zbody(i, c):
            acc_v[pl.ds(i * L, L)] = zeros
            return c

        lax.fori_loop(0, N // L, zbody, 0)

        ones = jnp.ones((L,), jnp.float32)

        def body(i, c):
            idx = row_v[pl.ds(i * L, L)]
            plsc.addupdate_scatter(acc_v, [idx], ones)
            return c

        lax.fori_loop(0, epw // L, body, 0)
        pltpu.sync_copy(acc_v, degp_hbm.at[wid])

    return hist


def _make_edges(E, N):
    nb = E // 128
    bpw = nb // NW
    ntail = nb - bpw * NW
    epw = bpw * 128
    mesh = plsc.VectorSubcoreMesh(core_axis_name="c", subcore_axis_name="s")

    @functools.partial(
        pl.kernel,
        mesh=mesh,
        out_type=(
            jax.ShapeDtypeStruct((nb, 2, 128), jnp.float32),
            jax.ShapeDtypeStruct((2 * NW, N), jnp.float32),
        ),
        compiler_params=pltpu.CompilerParams(needs_layout_passes=False),
        scratch_types=[
            pltpu.VMEM((epw + 128,), jnp.int32),
            pltpu.VMEM((epw + 128,), jnp.int32),
            pltpu.VMEM((N,), jnp.float32),
            pltpu.VMEM((N,), jnp.float32),
            pltpu.VMEM((N,), jnp.float32),
            pltpu.VMEM((N,), jnp.float32),
            pltpu.VMEM((bpw, 2, 128), jnp.float32),
            pltpu.VMEM((1, 2, 128), jnp.float32),
            pltpu.VMEM((N,), jnp.float32),
            pltpu.VMEM((N,), jnp.float32),
        ],
    )
    def edges(row_hbm, col_hbm, tab_hbm, att_hbm, sp_hbm,
              row_v, col_v, u_v, v_v, g0_v, g1_v, att_v, attt_v, s0_v, s1_v):
        wid = lax.axis_index("s") * NC + lax.axis_index("c")
        base = wid * epw
        pltpu.sync_copy(row_hbm.at[pl.ds(base, epw)], row_v.at[pl.ds(0, epw)])
        pltpu.sync_copy(col_hbm.at[pl.ds(base, epw)], col_v.at[pl.ds(0, epw)])

        tbase = (bpw * NW + wid) * 128
        @pl.when(wid < ntail)
        def _():
            pltpu.sync_copy(row_hbm.at[pl.ds(tbase, 128)],
                            row_v.at[pl.ds(epw, 128)])
            pltpu.sync_copy(col_hbm.at[pl.ds(tbase, 128)],
                            col_v.at[pl.ds(epw, 128)])

        pltpu.sync_copy(tab_hbm.at[0], u_v)
        pltpu.sync_copy(tab_hbm.at[1], v_v)
        pltpu.sync_copy(tab_hbm.at[2], g0_v)
        pltpu.sync_copy(tab_hbm.at[3], g1_v)

        zeros = jnp.zeros((L,), jnp.float32)

        def zbody(i, c):
            s0_v[pl.ds(i * L, L)] = zeros
            s1_v[pl.ds(i * L, L)] = zeros
            return c

        lax.fori_loop(0, N // L, zbody, 0)

        def block(b, att_ref, ab, ebase):
            for j in range(8):
                sl = pl.ds(ebase + j * L, L)
                idxr = row_v[sl]
                idxc = col_v[sl]
                ug = plsc.load_gather(u_v, [idxr])
                vg = plsc.load_gather(v_v, [idxc])
                e = jnp.exp(ug + vg)
                a0 = 1.0 / (1.0 + e)
                a1 = e * a0
                att_ref[ab, 0, pl.ds(j * L, L)] = a0
                att_ref[ab, 1, pl.ds(j * L, L)] = a1
                gr0 = plsc.load_gather(g0_v, [idxr])
                gr1 = plsc.load_gather(g1_v, [idxr])
                plsc.addupdate_scatter(s0_v, [idxc], gr0)
                plsc.addupdate_scatter(s1_v, [idxc], gr1)

        def body(b, c):
            block(b, att_v, b, b * 128)
            return c

        lax.fori_loop(0, bpw, body, 0)

        @pl.when(wid < ntail)
        def _():
            block(0, attt_v, 0, epw)

        pltpu.sync_copy(att_v, att_hbm.at[pl.ds(bpw * wid, bpw)])

        @pl.when(wid < ntail)
        def _():
            pltpu.sync_copy(attt_v, att_hbm.at[pl.ds(bpw * NW + wid, 1)])

        pltpu.sync_copy(s0_v, sp_hbm.at[wid])
        pltpu.sync_copy(s1_v, sp_hbm.at[NW + wid])

    return edges


def _tables_body(x_ref, wcat_ref, consts_ref, degp_ref, tab_ref, fin_ref):
    x = x_ref[...]
    wcat = wcat_ref[...]
    ht = lax.dot_general(wcat, x, (((0,), (1,)), ((), ())),
                         preferred_element_type=jnp.float32)
    c0 = consts_ref[0]
    bg0 = consts_ref[1]
    bg1 = consts_ref[2]
    deg = jnp.sum(degp_ref[...], axis=0, keepdims=True) + 1.0
    dis = lax.rsqrt(deg)
    u = ht[1:2] - ht[0:1] + c0
    v = ht[3:4] - ht[2:3]
    gp0 = dis * ht[4:5]
    gp1 = dis * ht[5:6]
    base0 = dis * gp0 + bg0
    base1 = dis * gp1 + bg1
    tab_ref[...] = jnp.concatenate([u, v, gp0, gp1], axis=0)
    fin_ref[...] = jnp.concatenate([dis, base0, base1], axis=0)


def _final_body(fin_ref, sp_ref, out_ref):
    dis = fin_ref[0:1]
    base0 = fin_ref[1:2]
    base1 = fin_ref[2:3]
    s0 = jnp.sum(sp_ref[0:NW], axis=0, keepdims=True)
    s1 = jnp.sum(sp_ref[NW:2 * NW], axis=0, keepdims=True)
    l0 = dis * s0 + base0
    l1 = dis * s1 + base1
    e = jnp.exp(l1 - l0)
    n0 = 1.0 / (1.0 + e)
    n1 = e * n0
    out_ref[...] = jnp.concatenate([n0, n1], axis=0)


def kernel(x, edge_index, W_edge, b_edge, W_gcn, b_gcn):
    N, D = x.shape
    E = edge_index.shape[1]

    wcat = jnp.concatenate([W_edge[:D], W_edge[D:], W_gcn], axis=1)
    consts = jnp.stack([b_edge[1] - b_edge[0], b_gcn[0], b_gcn[1]])

    row = edge_index[0]
    col = edge_index[1]
    degp = _make_hist(E, N)(row)

    tab, fin = pl.pallas_call(
        _tables_body,
        out_shape=(
            jax.ShapeDtypeStruct((4, N), jnp.float32),
            jax.ShapeDtypeStruct((3, N), jnp.float32),
        ),
        in_specs=[
            pl.BlockSpec(memory_space=pltpu.VMEM),
            pl.BlockSpec(memory_space=pltpu.VMEM),
            pl.BlockSpec(memory_space=pltpu.SMEM),
            pl.BlockSpec(memory_space=pltpu.VMEM),
        ],
    )(x, wcat, consts, degp)

    att3, sp = _make_edges(E, N)(row, col, tab)

    out2 = pl.pallas_call(
        _final_body,
        out_shape=jax.ShapeDtypeStruct((2, N), jnp.float32),
    )(fin, sp)

    edge_att = att3.transpose(0, 2, 1).reshape(E, 2)
    return edge_att, out2.T


# speedup vs baseline: 149.7485x; 1.3924x over previous
import functools

import jax
import jax.numpy as jnp
from jax import lax
from jax.experimental import pallas as pl
from jax.experimental.pallas import tpu as pltpu
from jax.experimental.pallas import tpu_sc as plsc

NC = 2
NS = 16
NW = NC * NS
L = 16


def _make_hist(E, N):
    nb = E // 128
    bpw = nb // NW
    ntail = nb - bpw * NW
    mesh = plsc.VectorSubcoreMesh(core_axis_name="c", subcore_axis_name="s")

    @functools.partial(
        pl.kernel,
        mesh=mesh,
        out_type=jax.ShapeDtypeStruct((NW, N), jnp.float32),
        compiler_params=pltpu.CompilerParams(needs_layout_passes=False),
        scratch_types=[
            pltpu.VMEM((bpw, 2, 128), jnp.int32),
            pltpu.VMEM((1, 2, 128), jnp.int32),
            pltpu.VMEM((N,), jnp.float32),
        ],
    )
    def hist(ei3_hbm, degp_hbm, rc_v, rct_v, acc_v):
        wid = lax.axis_index("s") * NC + lax.axis_index("c")
        pltpu.sync_copy(ei3_hbm.at[pl.ds(bpw * wid, bpw)], rc_v)

        @pl.when(wid < ntail)
        def _():
            pltpu.sync_copy(ei3_hbm.at[pl.ds(bpw * NW + wid, 1)], rct_v)

        zeros = jnp.zeros((L,), jnp.float32)

        def zbody(i, c):
            acc_v[pl.ds(i * L, L)] = zeros
            return c

        lax.fori_loop(0, N // L, zbody, 0)

        ones = jnp.ones((L,), jnp.float32)

        def body(b, c):
            for j in range(8):
                idx = rc_v[b, 0, pl.ds(j * L, L)]
                plsc.addupdate_scatter(acc_v, [idx], ones)
            return c

        lax.fori_loop(0, bpw, body, 0)

        @pl.when(wid < ntail)
        def _():
            for j in range(8):
                idx = rct_v[0, 0, pl.ds(j * L, L)]
                plsc.addupdate_scatter(acc_v, [idx], ones)

        pltpu.sync_copy(acc_v, degp_hbm.at[wid])

    return hist


def _make_edges(E, N):
    nb = E // 128
    bpw = nb // NW
    ntail = nb - bpw * NW
    mesh = plsc.VectorSubcoreMesh(core_axis_name="c", subcore_axis_name="s")

    @functools.partial(
        pl.kernel,
        mesh=mesh,
        out_type=(
            jax.ShapeDtypeStruct((nb, 2, 128), jnp.float32),
            jax.ShapeDtypeStruct((2 * NW, N), jnp.float32),
        ),
        compiler_params=pltpu.CompilerParams(needs_layout_passes=False),
        scratch_types=[
            pltpu.VMEM((bpw, 2, 128), jnp.int32),
            pltpu.VMEM((1, 2, 128), jnp.int32),
            pltpu.VMEM((N,), jnp.float32),
            pltpu.VMEM((N,), jnp.float32),
            pltpu.VMEM((N,), jnp.float32),
            pltpu.VMEM((N,), jnp.float32),
            pltpu.VMEM((bpw, 2, 128), jnp.float32),
            pltpu.VMEM((1, 2, 128), jnp.float32),
            pltpu.VMEM((N,), jnp.float32),
            pltpu.VMEM((N,), jnp.float32),
        ],
    )
    def edges(ei3_hbm, tab_hbm, att_hbm, sp_hbm,
              rc_v, rct_v, u_v, v_v, g0_v, g1_v, att_v, attt_v, s0_v, s1_v):
        wid = lax.axis_index("s") * NC + lax.axis_index("c")
        pltpu.sync_copy(ei3_hbm.at[pl.ds(bpw * wid, bpw)], rc_v)

        @pl.when(wid < ntail)
        def _():
            pltpu.sync_copy(ei3_hbm.at[pl.ds(bpw * NW + wid, 1)], rct_v)

        pltpu.sync_copy(tab_hbm.at[0], u_v)
        pltpu.sync_copy(tab_hbm.at[1], v_v)
        pltpu.sync_copy(tab_hbm.at[2], g0_v)
        pltpu.sync_copy(tab_hbm.at[3], g1_v)

        zeros = jnp.zeros((L,), jnp.float32)

        def zbody(i, c):
            s0_v[pl.ds(i * L, L)] = zeros
            s1_v[pl.ds(i * L, L)] = zeros
            return c

        lax.fori_loop(0, N // L, zbody, 0)

        def att_block(rc_ref, ab_in, att_ref, ab_out):
            for j in range(8):
                idxr = rc_ref[ab_in, 0, pl.ds(j * L, L)]
                idxc = rc_ref[ab_in, 1, pl.ds(j * L, L)]
                ug = plsc.load_gather(u_v, [idxr])
                vg = plsc.load_gather(v_v, [idxc])
                e = jnp.exp(ug + vg)
                a0 = 1.0 / (1.0 + e)
                a1 = e * a0
                att_ref[ab_out, 0, pl.ds(j * L, L)] = a0
                att_ref[ab_out, 1, pl.ds(j * L, L)] = a1

        def scat_block(rc_ref, ab_in):
            for j in range(8):
                idxr = rc_ref[ab_in, 0, pl.ds(j * L, L)]
                idxc = rc_ref[ab_in, 1, pl.ds(j * L, L)]
                gr0 = plsc.load_gather(g0_v, [idxr])
                gr1 = plsc.load_gather(g1_v, [idxr])
                plsc.addupdate_scatter(s0_v, [idxc], gr0)
                plsc.addupdate_scatter(s1_v, [idxc], gr1)

        @plsc.parallel_loop(0, bpw, 1, unroll=1)
        def _(b):
            att_block(rc_v, b, att_v, b)

        def sbody(b, c):
            scat_block(rc_v, b)
            return c

        lax.fori_loop(0, bpw, sbody, 0)

        @pl.when(wid < ntail)
        def _():
            att_block(rct_v, 0, attt_v, 0)
            scat_block(rct_v, 0)

        pltpu.sync_copy(att_v, att_hbm.at[pl.ds(bpw * wid, bpw)])

        @pl.when(wid < ntail)
        def _():
            pltpu.sync_copy(attt_v, att_hbm.at[pl.ds(bpw * NW + wid, 1)])

        pltpu.sync_copy(s0_v, sp_hbm.at[wid])
        pltpu.sync_copy(s1_v, sp_hbm.at[NW + wid])

    return edges


def _mm_body(x_ref, wcat_ref, ht_ref):
    ht_ref[...] = lax.dot_general(
        wcat_ref[...], x_ref[...], (((0,), (1,)), ((), ())),
        preferred_element_type=jnp.float32)


def _tables_body(ht_ref, consts_ref, degp_ref, tab_ref, fin_ref):
    ht = ht_ref[...]
    c0 = consts_ref[0]
    bg0 = consts_ref[1]
    bg1 = consts_ref[2]
    deg = jnp.sum(degp_ref[...], axis=0, keepdims=True) + 1.0
    dis = lax.rsqrt(deg)
    u = ht[1:2] - ht[0:1] + c0
    v = ht[3:4] - ht[2:3]
    gp0 = dis * ht[4:5]
    gp1 = dis * ht[5:6]
    base0 = dis * gp0 + bg0
    base1 = dis * gp1 + bg1
    tab_ref[...] = jnp.concatenate([u, v, gp0, gp1], axis=0)
    fin_ref[...] = jnp.concatenate([dis, base0, base1], axis=0)


def _final_body(fin_ref, sp_ref, out_ref):
    dis = fin_ref[0:1]
    base0 = fin_ref[1:2]
    base1 = fin_ref[2:3]
    s0 = jnp.sum(sp_ref[0:NW], axis=0, keepdims=True)
    s1 = jnp.sum(sp_ref[NW:2 * NW], axis=0, keepdims=True)
    l0 = dis * s0 + base0
    l1 = dis * s1 + base1
    e = jnp.exp(l1 - l0)
    n0 = 1.0 / (1.0 + e)
    n1 = e * n0
    out_ref[...] = jnp.concatenate([n0, n1], axis=0)


def kernel(x, edge_index, W_edge, b_edge, W_gcn, b_gcn):
    N, D = x.shape
    E = edge_index.shape[1]

    wcat = jnp.concatenate([W_edge[:D], W_edge[D:], W_gcn], axis=1)
    consts = jnp.stack([b_edge[1] - b_edge[0], b_gcn[0], b_gcn[1]])

    ei3 = edge_index.T.reshape(E // 128, 128, 2).transpose(0, 2, 1)

    degp = _make_hist(E, N)(ei3)

    ht = pl.pallas_call(
        _mm_body,
        out_shape=jax.ShapeDtypeStruct((6, N), jnp.float32),
    )(x, wcat)

    tab, fin = pl.pallas_call(
        _tables_body,
        out_shape=(
            jax.ShapeDtypeStruct((4, N), jnp.float32),
            jax.ShapeDtypeStruct((3, N), jnp.float32),
        ),
        in_specs=[
            pl.BlockSpec(memory_space=pltpu.VMEM),
            pl.BlockSpec(memory_space=pltpu.SMEM),
            pl.BlockSpec(memory_space=pltpu.VMEM),
        ],
    )(ht, consts, degp)

    att3, sp = _make_edges(E, N)(ei3, tab)

    out2 = pl.pallas_call(
        _final_body,
        out_shape=jax.ShapeDtypeStruct((2, N), jnp.float32),
    )(fin, sp)

    edge_att = att3.transpose(0, 2, 1).reshape(E, 2)
    return edge_att, out2.T
